# baseline (device time: 387169 ns/iter reference)
import functools

import jax
import jax.numpy as jnp
from jax import lax
from jax.experimental import pallas as pl
from jax.experimental.pallas import tpu as pltpu

N_DEV = 4


def _ag_body(k_ref, v_ref, kg_ref, vg_ref,
             copy_sems, ksend, krecv, vsend, vrecv):
    my = lax.axis_index("i")
    left = lax.rem(my + N_DEV - 1, N_DEV)
    right = lax.rem(my + 1, N_DEV)

    barrier_sem = pltpu.get_barrier_semaphore()
    for nbr in (left, right):
        pl.semaphore_signal(
            barrier_sem, inc=1,
            device_id=(nbr,), device_id_type=pl.DeviceIdType.MESH,
        )
    pl.semaphore_wait(barrier_sem, 2)

    kcopy = pltpu.make_async_copy(k_ref, kg_ref.at[my], copy_sems.at[0])
    vcopy = pltpu.make_async_copy(v_ref, vg_ref.at[my], copy_sems.at[1])
    kcopy.start()
    vcopy.start()

    for h in range(N_DEV - 1):
        k_origin = lax.rem(my + N_DEV - h, N_DEV)
        v_origin = lax.rem(my + h, N_DEV)
        k_src = k_ref if h == 0 else kg_ref.at[k_origin]
        v_src = v_ref if h == 0 else vg_ref.at[v_origin]
        k_rdma = pltpu.make_async_remote_copy(
            src_ref=k_src,
            dst_ref=kg_ref.at[k_origin],
            send_sem=ksend.at[h],
            recv_sem=krecv.at[h],
            device_id=(right,),
            device_id_type=pl.DeviceIdType.MESH,
        )
        v_rdma = pltpu.make_async_remote_copy(
            src_ref=v_src,
            dst_ref=vg_ref.at[v_origin],
            send_sem=vsend.at[h],
            recv_sem=vrecv.at[h],
            device_id=(left,),
            device_id_type=pl.DeviceIdType.MESH,
        )
        k_rdma.start()
        v_rdma.start()
        k_rdma.wait()
        v_rdma.wait()

    kcopy.wait()
    vcopy.wait()


def _all_gather_kv(k_shard, v_shard):
    b, s, h, d = k_shard.shape
    gathered = jax.ShapeDtypeStruct((N_DEV, b, s, h, d), k_shard.dtype)
    return pl.pallas_call(
        _ag_body,
        out_shape=(gathered, gathered),
        in_specs=[
            pl.BlockSpec(memory_space=pl.ANY),
            pl.BlockSpec(memory_space=pl.ANY),
        ],
        out_specs=(
            pl.BlockSpec(memory_space=pl.ANY),
            pl.BlockSpec(memory_space=pl.ANY),
        ),
        scratch_shapes=[
            pltpu.SemaphoreType.DMA((2,)),
            pltpu.SemaphoreType.DMA((N_DEV - 1,)),
            pltpu.SemaphoreType.DMA((N_DEV - 1,)),
            pltpu.SemaphoreType.DMA((N_DEV - 1,)),
            pltpu.SemaphoreType.DMA((N_DEV - 1,)),
        ],
        compiler_params=pltpu.CompilerParams(collective_id=0),
    )(k_shard, v_shard)


def kernel(Q, K, V):
    kg, vg = _all_gather_kv(K, V)

    b, sq, nh, d = Q.shape
    scale = d ** -0.5
    s5 = jnp.einsum("bqhd,obkhd->bhqok", Q, kg) * scale
    s_flat = s5.reshape(b, nh, sq, N_DEV * sq)
    m = s_flat.max(-1, keepdims=True)
    p = jnp.exp(s_flat - m)
    p = p / p.sum(-1, keepdims=True)
    p5 = p.reshape(b, nh, sq, N_DEV, sq)
    return jnp.einsum("bhqok,obkhd->bqhd", p5, vg).astype(jnp.float32)


# device time: 164210 ns/iter; 2.3578x vs baseline; 2.3578x over previous
import jax
import jax.numpy as jnp
from jax import lax
from jax.experimental import pallas as pl
from jax.experimental.pallas import tpu as pltpu

N_DEV = 4
N_HOP = N_DEV - 1


def _flash_update(q_ref, out_ref, m_ref, l_ref, kc, vc, rows, first):
    R = kc.shape[0]
    rs = pl.ds(rows, R)
    scale = q_ref.shape[1] ** -0.5
    q = q_ref[rs] * scale
    s = lax.dot_general(q, kc, (((1,), (1,)), ((0,), (0,))))
    mc = jnp.max(s, axis=-1)
    if first:
        m_new = mc
        p = jnp.exp(s - m_new[:, :, None])
        l_new = jnp.sum(p, axis=-1)
        pv = lax.dot_general(vc, p, (((2,), (2,)), ((0,), (0,))))
        out_ref[rs] = pv
    else:
        m_old = m_ref[rs]
        m_new = jnp.maximum(m_old, mc)
        alpha = jnp.exp(m_old - m_new)
        p = jnp.exp(s - m_new[:, :, None])
        l_new = alpha * l_ref[rs] + jnp.sum(p, axis=-1)
        pv = lax.dot_general(vc, p, (((2,), (2,)), ((0,), (0,))))
        out_ref[rs] = out_ref[rs] * alpha[:, None, :] + pv
    m_ref[rs] = m_new
    l_ref[rs] = l_new


def _body(q_ref, k_ref, v_ref, out_ref,
          cwk, cwv, ccwk, ccwv, m_ref, l_ref,
          cwk_s, cwk_r, cwv_s, cwv_r, ccwk_s, ccwk_r, ccwv_s, ccwv_r):
    my = lax.axis_index("i")
    left = lax.rem(my + N_DEV - 1, N_DEV)
    right = lax.rem(my + 1, N_DEV)
    bh = q_ref.shape[0]
    half = bh // 2

    barrier_sem = pltpu.get_barrier_semaphore()
    for nbr in (left, right):
        pl.semaphore_signal(
            barrier_sem, inc=1,
            device_id=(nbr,), device_id_type=pl.DeviceIdType.MESH,
        )
    pl.semaphore_wait(barrier_sem, 2)

    recvs = []
    for h in range(N_HOP):
        hops = []
        for ring, kbuf, vbuf, ks, kr, vs, vr, dst_dev in (
            ("cw", cwk, cwv, cwk_s, cwk_r, cwv_s, cwv_r, right),
            ("ccw", ccwk, ccwv, ccwk_s, ccwk_r, ccwv_s, ccwv_r, left),
        ):
            row0 = 0 if ring == "cw" else half
            k_src = k_ref.at[pl.ds(row0, half)] if h == 0 else kbuf.at[h - 1]
            v_src = v_ref.at[pl.ds(row0, half)] if h == 0 else vbuf.at[h - 1]
            for src, buf, ssem, rsem in (
                (k_src, kbuf, ks, kr),
                (v_src, vbuf, vs, vr),
            ):
                rdma = pltpu.make_async_remote_copy(
                    src_ref=src, dst_ref=buf.at[h],
                    send_sem=ssem.at[h], recv_sem=rsem.at[h],
                    device_id=(dst_dev,),
                    device_id_type=pl.DeviceIdType.MESH,
                )
                rdma.start()
                hops.append(rdma)

        if h == 0:
            _flash_update(q_ref, out_ref, m_ref, l_ref,
                          k_ref[pl.ds(0, half)], v_ref[pl.ds(0, half)],
                          0, first=True)
            _flash_update(q_ref, out_ref, m_ref, l_ref,
                          k_ref[pl.ds(half, half)], v_ref[pl.ds(half, half)],
                          half, first=True)
        else:
            _flash_update(q_ref, out_ref, m_ref, l_ref,
                          cwk[h - 1], cwv[h - 1], 0, first=False)
            _flash_update(q_ref, out_ref, m_ref, l_ref,
                          ccwk[h - 1], ccwv[h - 1], half, first=False)

        for rdma in hops:
            rdma.wait_recv()
        recvs.extend(hops)

    _flash_update(q_ref, out_ref, m_ref, l_ref,
                  cwk[N_HOP - 1], cwv[N_HOP - 1], 0, first=False)
    _flash_update(q_ref, out_ref, m_ref, l_ref,
                  ccwk[N_HOP - 1], ccwv[N_HOP - 1], half, first=False)
    out_ref[:] = out_ref[:] / l_ref[:][:, None, :]

    for rdma in recvs:
        rdma.wait_send()


def _ring_attn(q3, k3, v3):
    bh, d, s = q3.shape
    half = bh // 2
    comm = lambda: pltpu.VMEM((N_HOP, half, d, s), q3.dtype)
    sem3 = lambda: pltpu.SemaphoreType.DMA((N_HOP,))
    return pl.pallas_call(
        _body,
        out_shape=jax.ShapeDtypeStruct((bh, d, s), q3.dtype),
        in_specs=[pl.BlockSpec(memory_space=pltpu.VMEM)] * 3,
        out_specs=pl.BlockSpec(memory_space=pltpu.VMEM),
        scratch_shapes=[
            comm(), comm(), comm(), comm(),
            pltpu.VMEM((bh, s), q3.dtype),
            pltpu.VMEM((bh, s), q3.dtype),
            sem3(), sem3(), sem3(), sem3(),
            sem3(), sem3(), sem3(), sem3(),
        ],
        compiler_params=pltpu.CompilerParams(
            collective_id=0, vmem_limit_bytes=100 * 1024 * 1024,
        ),
    )(q3, k3, v3)


def kernel(Q, K, V):
    b, s, h, d = Q.shape
    to3 = lambda x: x.transpose(0, 2, 3, 1).reshape(b * h, d, s)
    o3 = _ring_attn(to3(Q), to3(K), to3(V))
    return o3.reshape(b, h, d, s).transpose(0, 3, 1, 2).astype(jnp.float32)


# device time: 101187 ns/iter; 3.8263x vs baseline; 1.6228x over previous
import jax
import jax.numpy as jnp
from jax import lax
from jax.experimental import pallas as pl
from jax.experimental.pallas import tpu as pltpu

N_DEV = 4
N_HOP = N_DEV - 1


def _flash_update(q_ref, out_ref, m_ref, l_ref, kc, vc, rows, first):
    R = kc.shape[0]
    rs = pl.ds(rows, R)
    scale = q_ref.shape[1] ** -0.5
    q = (q_ref[rs] * scale).astype(kc.dtype)
    s = lax.dot_general(q, kc, (((1,), (1,)), ((0,), (0,))),
                        preferred_element_type=jnp.float32)
    mc = jnp.max(s, axis=-1)
    if first:
        m_new = mc
        p = jnp.exp(s - m_new[:, :, None])
        l_new = jnp.sum(p, axis=-1)
        pv = lax.dot_general(vc, p.astype(vc.dtype),
                             (((2,), (2,)), ((0,), (0,))),
                             preferred_element_type=jnp.float32)
        out_ref[rs] = pv
    else:
        m_old = m_ref[rs]
        m_new = jnp.maximum(m_old, mc)
        alpha = jnp.exp(m_old - m_new)
        p = jnp.exp(s - m_new[:, :, None])
        l_new = alpha * l_ref[rs] + jnp.sum(p, axis=-1)
        pv = lax.dot_general(vc, p.astype(vc.dtype),
                             (((2,), (2,)), ((0,), (0,))),
                             preferred_element_type=jnp.float32)
        out_ref[rs] = out_ref[rs] * alpha[:, None, :] + pv
    m_ref[rs] = m_new
    l_ref[rs] = l_new


def _body(q_ref, k_ref, v_ref, out_ref,
          cwk, cwv, ccwk, ccwv, m_ref, l_ref,
          cwk_s, cwk_r, cwv_s, cwv_r, ccwk_s, ccwk_r, ccwv_s, ccwv_r):
    my = lax.axis_index("i")
    left = lax.rem(my + N_DEV - 1, N_DEV)
    right = lax.rem(my + 1, N_DEV)
    bh = q_ref.shape[0]
    half = bh // 2

    barrier_sem = pltpu.get_barrier_semaphore()
    for nbr in (left, right):
        pl.semaphore_signal(
            barrier_sem, inc=1,
            device_id=(nbr,), device_id_type=pl.DeviceIdType.MESH,
        )
    pl.semaphore_wait(barrier_sem, 2)

    recvs = []
    for h in range(N_HOP):
        hops = []
        for ring, kbuf, vbuf, ks, kr, vs, vr, dst_dev in (
            ("cw", cwk, cwv, cwk_s, cwk_r, cwv_s, cwv_r, right),
            ("ccw", ccwk, ccwv, ccwk_s, ccwk_r, ccwv_s, ccwv_r, left),
        ):
            row0 = 0 if ring == "cw" else half
            k_src = k_ref.at[pl.ds(row0, half)] if h == 0 else kbuf.at[h - 1]
            v_src = v_ref.at[pl.ds(row0, half)] if h == 0 else vbuf.at[h - 1]
            for src, buf, ssem, rsem in (
                (k_src, kbuf, ks, kr),
                (v_src, vbuf, vs, vr),
            ):
                rdma = pltpu.make_async_remote_copy(
                    src_ref=src, dst_ref=buf.at[h],
                    send_sem=ssem.at[h], recv_sem=rsem.at[h],
                    device_id=(dst_dev,),
                    device_id_type=pl.DeviceIdType.MESH,
                )
                rdma.start()
                hops.append(rdma)

        if h == 0:
            _flash_update(q_ref, out_ref, m_ref, l_ref,
                          k_ref[pl.ds(0, half)], v_ref[pl.ds(0, half)],
                          0, first=True)
            _flash_update(q_ref, out_ref, m_ref, l_ref,
                          k_ref[pl.ds(half, half)], v_ref[pl.ds(half, half)],
                          half, first=True)
        else:
            _flash_update(q_ref, out_ref, m_ref, l_ref,
                          cwk[h - 1], cwv[h - 1], 0, first=False)
            _flash_update(q_ref, out_ref, m_ref, l_ref,
                          ccwk[h - 1], ccwv[h - 1], half, first=False)

        for rdma in hops:
            rdma.wait_recv()
        recvs.extend(hops)

    _flash_update(q_ref, out_ref, m_ref, l_ref,
                  cwk[N_HOP - 1], cwv[N_HOP - 1], 0, first=False)
    _flash_update(q_ref, out_ref, m_ref, l_ref,
                  ccwk[N_HOP - 1], ccwv[N_HOP - 1], half, first=False)
    out_ref[:] = out_ref[:] / l_ref[:][:, None, :]

    for rdma in recvs:
        rdma.wait_send()


def _ring_attn(q3, k3, v3):
    bh, d, s = q3.shape
    half = bh // 2
    comm = lambda: pltpu.VMEM((N_HOP, half, d, s), k3.dtype)
    sem3 = lambda: pltpu.SemaphoreType.DMA((N_HOP,))
    return pl.pallas_call(
        _body,
        out_shape=jax.ShapeDtypeStruct((bh, d, s), q3.dtype),
        in_specs=[pl.BlockSpec(memory_space=pltpu.VMEM)] * 3,
        out_specs=pl.BlockSpec(memory_space=pltpu.VMEM),
        scratch_shapes=[
            comm(), comm(), comm(), comm(),
            pltpu.VMEM((bh, s), q3.dtype),
            pltpu.VMEM((bh, s), q3.dtype),
            sem3(), sem3(), sem3(), sem3(),
            sem3(), sem3(), sem3(), sem3(),
        ],
        compiler_params=pltpu.CompilerParams(
            collective_id=0, vmem_limit_bytes=100 * 1024 * 1024,
        ),
    )(q3, k3, v3)


def kernel(Q, K, V):
    b, s, h, d = Q.shape
    to3 = lambda x: x.transpose(0, 2, 3, 1).reshape(b * h, d, s)
    o3 = _ring_attn(to3(Q),
                    to3(K).astype(jnp.bfloat16),
                    to3(V).astype(jnp.bfloat16))
    return o3.reshape(b, h, d, s).transpose(0, 3, 1, 2).astype(jnp.float32)


# device time: 99132 ns/iter; 3.9056x vs baseline; 1.0207x over previous
import jax
import jax.numpy as jnp
from jax import lax
from jax.experimental import pallas as pl
from jax.experimental.pallas import tpu as pltpu

N_DEV = 4
N_HOP = N_DEV - 1


def _flash_update(q_ref, out_ref, m_ref, l_ref, kc, vc, rows, first):
    R = kc.shape[0]
    rs = pl.ds(rows, R)
    q = q_ref[rs]
    s = lax.dot_general(q, kc, (((1,), (1,)), ((0,), (0,))),
                        preferred_element_type=jnp.float32)
    mc = jnp.max(s, axis=-1)
    if first:
        m_new = mc
        p = jnp.exp(s - m_new[:, :, None])
        l_new = jnp.sum(p, axis=-1)
        pv = lax.dot_general(vc, p.astype(vc.dtype),
                             (((2,), (2,)), ((0,), (0,))),
                             preferred_element_type=jnp.float32)
        out_ref[rs] = pv
    else:
        m_old = m_ref[rs]
        m_new = jnp.maximum(m_old, mc)
        alpha = jnp.exp(m_old - m_new)
        p = jnp.exp(s - m_new[:, :, None])
        l_new = alpha * l_ref[rs] + jnp.sum(p, axis=-1)
        pv = lax.dot_general(vc, p.astype(vc.dtype),
                             (((2,), (2,)), ((0,), (0,))),
                             preferred_element_type=jnp.float32)
        out_ref[rs] = out_ref[rs] * alpha[:, None, :] + pv
    m_ref[rs] = m_new
    l_ref[rs] = l_new


def _body(q_ref, k_ref, v_ref, out_ref,
          cwk, cwv, ccwk, ccwv, m_ref, l_ref,
          cwk_s, cwk_r, cwv_s, cwv_r, ccwk_s, ccwk_r, ccwv_s, ccwv_r):
    my = lax.axis_index("i")
    left = lax.rem(my + N_DEV - 1, N_DEV)
    right = lax.rem(my + 1, N_DEV)
    bh = q_ref.shape[0]
    half = bh // 2
    quarter = half // 2

    barrier_sem = pltpu.get_barrier_semaphore()
    for nbr in (left, right):
        pl.semaphore_signal(
            barrier_sem, inc=1,
            device_id=(nbr,), device_id_type=pl.DeviceIdType.MESH,
        )
    pl.semaphore_wait(barrier_sem, 2)

    def _rdma(src, dst, ssem, rsem, dev):
        r = pltpu.make_async_remote_copy(
            src_ref=src, dst_ref=dst, send_sem=ssem, recv_sem=rsem,
            device_id=(dev,), device_id_type=pl.DeviceIdType.MESH,
        )
        r.start()
        return r

    done = []
    for h in (0, 1):
        if h == 0:
            cw_k, cw_v = k_ref.at[pl.ds(0, half)], v_ref.at[pl.ds(0, half)]
            ccw_k, ccw_v = k_ref.at[pl.ds(half, half)], v_ref.at[pl.ds(half, half)]
        else:
            cw_k, cw_v = cwk.at[h - 1], cwv.at[h - 1]
            ccw_k, ccw_v = ccwk.at[h - 1], ccwv.at[h - 1]
        batch = [
            _rdma(cw_k, cwk.at[h], cwk_s.at[h], cwk_r.at[h], right),
            _rdma(cw_v, cwv.at[h], cwv_s.at[h], cwv_r.at[h], right),
            _rdma(ccw_k, ccwk.at[h], ccwk_s.at[h], ccwk_r.at[h], left),
            _rdma(ccw_v, ccwv.at[h], ccwv_s.at[h], ccwv_r.at[h], left),
        ]
        if h == 0:
            _flash_update(q_ref, out_ref, m_ref, l_ref,
                          k_ref[pl.ds(0, half)], v_ref[pl.ds(0, half)],
                          0, first=True)
            _flash_update(q_ref, out_ref, m_ref, l_ref,
                          k_ref[pl.ds(half, half)], v_ref[pl.ds(half, half)],
                          half, first=True)
        else:
            _flash_update(q_ref, out_ref, m_ref, l_ref,
                          cwk[0], cwv[0], 0, first=False)
            _flash_update(q_ref, out_ref, m_ref, l_ref,
                          ccwk[0], ccwv[0], half, first=False)
        for r in batch:
            r.wait_recv()
        done += batch

    subs = []
    for i in (0, 1):
        rs = pl.ds(i * quarter, quarter)
        sm = N_HOP - 1 + i
        subs.append([
            _rdma(cwk.at[1, rs], cwk.at[2, rs], cwk_s.at[sm], cwk_r.at[sm], right),
            _rdma(cwv.at[1, rs], cwv.at[2, rs], cwv_s.at[sm], cwv_r.at[sm], right),
            _rdma(ccwk.at[1, rs], ccwk.at[2, rs], ccwk_s.at[sm], ccwk_r.at[sm], left),
            _rdma(ccwv.at[1, rs], ccwv.at[2, rs], ccwv_s.at[sm], ccwv_r.at[sm], left),
        ])
    _flash_update(q_ref, out_ref, m_ref, l_ref, cwk[1], cwv[1], 0, first=False)
    _flash_update(q_ref, out_ref, m_ref, l_ref, ccwk[1], ccwv[1], half,
                  first=False)
    for i in (0, 1):
        for r in subs[i]:
            r.wait_recv()
        sl = pl.ds(i * quarter, quarter)
        _flash_update(q_ref, out_ref, m_ref, l_ref,
                      cwk[2, sl], cwv[2, sl], i * quarter, first=False)
        _flash_update(q_ref, out_ref, m_ref, l_ref,
                      ccwk[2, sl], ccwv[2, sl], half + i * quarter, first=False)
        done += subs[i]

    out_ref[:] = out_ref[:] / l_ref[:][:, None, :]

    for r in done:
        r.wait_send()


def _ring_attn(q3, k3, v3):
    bh, d, s = q3.shape
    half = bh // 2
    comm = lambda: pltpu.VMEM((N_HOP, half, d, s), k3.dtype)
    sem = lambda: pltpu.SemaphoreType.DMA((N_HOP + 1,))
    return pl.pallas_call(
        _body,
        out_shape=jax.ShapeDtypeStruct((bh, d, s), jnp.float32),
        in_specs=[pl.BlockSpec(memory_space=pltpu.VMEM)] * 3,
        out_specs=pl.BlockSpec(memory_space=pltpu.VMEM),
        scratch_shapes=[
            comm(), comm(), comm(), comm(),
            pltpu.VMEM((bh, s), jnp.float32),
            pltpu.VMEM((bh, s), jnp.float32),
            sem(), sem(), sem(), sem(),
            sem(), sem(), sem(), sem(),
        ],
        compiler_params=pltpu.CompilerParams(
            collective_id=0, vmem_limit_bytes=100 * 1024 * 1024,
        ),
    )(q3, k3, v3)


def kernel(Q, K, V):
    b, s, h, d = Q.shape
    to3 = lambda x: x.transpose(0, 2, 3, 1).reshape(b * h, d, s)
    scale = d ** -0.5
    o3 = _ring_attn(to3(Q * scale).astype(jnp.bfloat16),
                    to3(K).astype(jnp.bfloat16),
                    to3(V).astype(jnp.bfloat16))
    return o3.reshape(b, h, d, s).transpose(0, 3, 1, 2)
